# baseline (speedup 1.0000x reference)
import functools

import jax
import jax.numpy as jnp
from jax import lax
from jax.experimental import pallas as pl
from jax.experimental.pallas import tpu as pltpu
from jax.experimental.pallas import tpu_sc as plsc

EMB_DIM = 64
BATCH = 4096
HIST = 200
TOTAL = BATCH * HIST

_INFO = plsc.get_sparse_core_info()
NC = _INFO.num_cores
NS = _INFO.num_subcores
NW = NC * NS
PER_W = TOTAL // NW
CHUNK = 128
NCH = PER_W // CHUNK
K = 4
NROUND = NCH // (2 * K)


def _gather_body(idx_hbm, table_hbm, out_hbm,
                 idx_v, buf_a, buf_b, sem_ag, sem_as, sem_bg, sem_bs):
    c = lax.axis_index("c")
    s = lax.axis_index("s")
    wid = s * NC + c
    base_chunk = wid * NCH

    pltpu.sync_copy(idx_hbm.at[pl.ds(base_chunk, NCH)], idx_v)

    def fire_gathers(buf, sem, group):
        for k in range(K):
            pltpu.async_copy(
                table_hbm.at[idx_v.at[group * K + k]],
                buf.at[k],
                sem,
            )

    def wait_gathers(buf, sem):
        for k in range(K):
            pltpu.make_async_copy(
                table_hbm.at[idx_v.at[0]], buf.at[k], sem,
            ).wait()

    def fire_scatters(buf, sem, group):
        pltpu.async_copy(
            buf,
            out_hbm.at[pl.ds(base_chunk + group * K, K)],
            sem,
        )

    def wait_scatters(buf, sem):
        pltpu.make_async_copy(
            buf, out_hbm.at[pl.ds(base_chunk, K)], sem,
        ).wait()

    fire_gathers(buf_a, sem_ag, 0)

    def round_body(r, carry):
        fire_gathers(buf_b, sem_bg, 2 * r + 1)
        wait_gathers(buf_a, sem_ag)
        fire_scatters(buf_a, sem_as, 2 * r)
        wait_scatters(buf_a, sem_as)

        @pl.when(r + 1 < NROUND)
        def _():
            fire_gathers(buf_a, sem_ag, 2 * r + 2)

        wait_gathers(buf_b, sem_bg)
        fire_scatters(buf_b, sem_bs, 2 * r + 1)
        wait_scatters(buf_b, sem_bs)
        return carry

    lax.fori_loop(0, NROUND, round_body, 0)


@functools.partial(
    pl.kernel,
    out_type=jax.ShapeDtypeStruct((TOTAL // CHUNK, CHUNK, EMB_DIM), jnp.float32),
    mesh=plsc.VectorSubcoreMesh(core_axis_name="c", subcore_axis_name="s"),
    scratch_types=[
        pltpu.VMEM((NCH, CHUNK), jnp.int32),
        pltpu.VMEM((K, CHUNK, EMB_DIM), jnp.float32),
        pltpu.VMEM((K, CHUNK, EMB_DIM), jnp.float32),
        pltpu.SemaphoreType.DMA,
        pltpu.SemaphoreType.DMA,
        pltpu.SemaphoreType.DMA,
        pltpu.SemaphoreType.DMA,
    ],
    compiler_params=pltpu.CompilerParams(use_tc_tiling_on_sc=False),
)
def _gather_kernel(idx_hbm, table_hbm, out_hbm,
                   idx_v, buf_a, buf_b, sem_ag, sem_as, sem_bg, sem_bs):
    _gather_body(idx_hbm, table_hbm, out_hbm,
                 idx_v, buf_a, buf_b, sem_ag, sem_as, sem_bg, sem_bs)


def kernel(indices, table):
    idx2d = indices.reshape(TOTAL // CHUNK, CHUNK)
    out = _gather_kernel(idx2d, table)
    return out.reshape(BATCH, HIST, EMB_DIM)
